# Initial kernel scaffold; baseline (speedup 1.0000x reference)
import functools

import jax
import jax.numpy as jnp
from jax import lax
from jax.experimental import pallas as pl
from jax.experimental.pallas import tpu as pltpu
from jax.experimental.pallas import tpu_sc as plsc

N = 10000
E = 320000
D = 128
ALPHA = 0.2

NC = 2
NS = 16
NW = NC * NS
CH = 128
CPT = 79
EPAD = NW * CPT * CH
NP = 10240
NPT = NP // NS

_mesh = plsc.VectorSubcoreMesh(core_axis_name="c", subcore_axis_name="s")


@functools.partial(
    pl.kernel,
    out_type=jax.ShapeDtypeStruct((NC, NP), jnp.float32),
    mesh=_mesh,
    scratch_types=[
        pltpu.VMEM((CPT, CH), jnp.int32),
        pltpu.VMEM((CH,), jnp.float32),
        pltpu.VMEM((NPT,), jnp.float32),
        pltpu.VMEM_SHARED((NP,), jnp.float32),
        pltpu.SemaphoreType.DMA,
    ],
)
def _deg_kernel(dst_hbm, out_hbm, dst_v, ones_v, zeros_v, deg_sh, sem):
    c = lax.axis_index("c")
    s = lax.axis_index("s")
    wid = c * NS + s
    for i in range(CH // 16):
        ones_v[pl.ds(i * 16, 16)] = jnp.ones((16,), jnp.float32)
    for i in range(NPT // 16):
        zeros_v[pl.ds(i * 16, 16)] = jnp.zeros((16,), jnp.float32)
    pltpu.sync_copy(zeros_v, deg_sh.at[pl.ds(s * NPT, NPT)])
    pltpu.async_copy(dst_hbm.at[wid], dst_v, sem).wait()
    plsc.subcore_barrier()

    def body(j, carry):
        pltpu.sync_copy(ones_v, deg_sh.at[dst_v.at[j]], add=True)
        return carry

    lax.fori_loop(0, CPT, body, 0)
    plsc.subcore_barrier()
    pltpu.sync_copy(deg_sh.at[pl.ds(s * NPT, NPT)],
                    out_hbm.at[c, pl.ds(s * NPT, NPT)])


@functools.partial(
    pl.kernel,
    out_type=jax.ShapeDtypeStruct((NC, NP, D), jnp.float32),
    mesh=_mesh,
    scratch_types=[
        pltpu.VMEM((CPT, CH), jnp.int32),
        pltpu.VMEM((CPT, CH), jnp.int32),
        pltpu.VMEM((CH, D), jnp.float32),
        pltpu.VMEM((16, D), jnp.float32),
        pltpu.VMEM_SHARED((NP, D), jnp.float32),
        pltpu.SemaphoreType.DMA,
    ],
)
def _agg_kernel(y_hbm, src_hbm, dst_hbm, out_hbm,
                src_v, dst_v, rows_v, zeros_v, agg_sh, sem):
    c = lax.axis_index("c")
    s = lax.axis_index("s")
    wid = c * NS + s
    for i in range(16):
        for k in range(D // 16):
            zeros_v[i, pl.ds(k * 16, 16)] = jnp.zeros((16,), jnp.float32)

    def zbody(j, carry):
        pltpu.sync_copy(zeros_v, agg_sh.at[pl.ds(s * NPT + j * 16, 16)])
        return carry

    lax.fori_loop(0, NPT // 16, zbody, 0)
    pltpu.sync_copy(src_hbm.at[wid], src_v)
    pltpu.sync_copy(dst_hbm.at[wid], dst_v)
    plsc.subcore_barrier()

    def body(j, carry):
        pltpu.async_copy(y_hbm.at[src_v.at[j]], rows_v, sem).wait()
        pltpu.sync_copy(rows_v, agg_sh.at[dst_v.at[j]], add=True)
        return carry

    lax.fori_loop(0, CPT, body, 0)
    plsc.subcore_barrier()

    def wbody(j, carry):
        pltpu.sync_copy(agg_sh.at[pl.ds(s * NPT + j * 128, 128)],
                        out_hbm.at[c, pl.ds(s * NPT + j * 128, 128)])
        return carry

    lax.fori_loop(0, NPT // 128, wbody, 0)


def _prep_body(degp_ref, x_ref, dinv_ref, y_ref):
    d = degp_ref[0, :] + degp_ref[1, :] + 1.0
    dinv = lax.rsqrt(d).reshape(-1, 1)
    dinv_ref[...] = dinv
    y_ref[...] = x_ref[...] * dinv


def _prep(degp, xp):
    blk = 128
    grid = NP // blk
    return pl.pallas_call(
        _prep_body,
        grid=(grid,),
        in_specs=[
            pl.BlockSpec((NC, blk), lambda i: (0, i)),
            pl.BlockSpec((blk, D), lambda i: (i, 0)),
        ],
        out_specs=[
            pl.BlockSpec((blk, 1), lambda i: (i, 0)),
            pl.BlockSpec((blk, D), lambda i: (i, 0)),
        ],
        out_shape=[
            jax.ShapeDtypeStruct((NP, 1), jnp.float32),
            jax.ShapeDtypeStruct((NP, D), jnp.float32),
        ],
    )(degp, xp)


def _final_body(p_ref, dinv_ref, y_ref, x0_ref, x_ref, w_ref, o_ref):
    ssum = p_ref[0] + p_ref[1] + y_ref[...]
    agg = ssum * dinv_ref[...]
    h = (1.0 - ALPHA) * agg + ALPHA * x0_ref[...]
    mm = jnp.dot(h, w_ref[...], preferred_element_type=jnp.float32)
    o_ref[...] = x_ref[...] + jnp.maximum(mm, 0.0)


def _final(parts, dinv, y, x0p, xp, W1):
    blk = 512
    grid = NP // blk
    return pl.pallas_call(
        _final_body,
        grid=(grid,),
        in_specs=[
            pl.BlockSpec((NC, blk, D), lambda i: (0, i, 0)),
            pl.BlockSpec((blk, 1), lambda i: (i, 0)),
            pl.BlockSpec((blk, D), lambda i: (i, 0)),
            pl.BlockSpec((blk, D), lambda i: (i, 0)),
            pl.BlockSpec((blk, D), lambda i: (i, 0)),
            pl.BlockSpec((D, D), lambda i: (0, 0)),
        ],
        out_specs=pl.BlockSpec((blk, D), lambda i: (i, 0)),
        out_shape=jax.ShapeDtypeStruct((NP, D), jnp.float32),
    )(parts, dinv, y, x0p, xp, W1)


def kernel(x, x0, edge_index, W1):
    src = edge_index[0]
    dst = edge_index[1]
    pad = jnp.full((EPAD - E,), N, jnp.int32)
    srcp = jnp.concatenate([src, pad]).reshape(NW, CPT, CH)
    dstp = jnp.concatenate([dst, pad]).reshape(NW, CPT, CH)
    xp = jnp.pad(x, ((0, NP - N), (0, 0)))
    x0p = jnp.pad(x0, ((0, NP - N), (0, 0)))

    degp = _deg_kernel(dstp)
    dinv, y = _prep(degp, xp)
    parts = _agg_kernel(y, srcp, dstp)
    outp = _final(parts, dinv, y, x0p, xp, W1)
    return outp[:N]


# speedup vs baseline: 14.9743x; 14.9743x over previous
import functools

import jax
import jax.numpy as jnp
from jax import lax
from jax.experimental import pallas as pl
from jax.experimental.pallas import tpu as pltpu
from jax.experimental.pallas import tpu_sc as plsc

N = 10000
E = 320000
D = 128
ALPHA = 0.2

NC = 2
NS = 16
NW = NC * NS
CH = 128
CPT = 79
EPAD = NW * CPT * CH
NP = 10240
NPT = NP // NS

def _deg_body(dst_hbm, out_hbm, dst_v, ones_v, zeros_v, deg_sh, sem):
    c = lax.axis_index("c")
    s = lax.axis_index("s")
    wid = c * NS + s
    for i in range(CH // 16):
        ones_v[pl.ds(i * 16, 16)] = jnp.ones((16,), jnp.float32)
    for i in range(NPT // 16):
        zeros_v[pl.ds(i * 16, 16)] = jnp.zeros((16,), jnp.float32)
    pltpu.sync_copy(zeros_v, deg_sh.at[pl.ds(s * NPT, NPT)])
    pltpu.async_copy(dst_hbm.at[wid], dst_v, sem).wait()
    plsc.subcore_barrier()

    def body(j, carry):
        pltpu.sync_copy(ones_v, deg_sh.at[dst_v.at[j]], add=True)
        return carry

    lax.fori_loop(0, CPT, body, 0)
    plsc.subcore_barrier()
    pltpu.sync_copy(deg_sh.at[pl.ds(s * NPT, NPT)],
                    out_hbm.at[c, pl.ds(s * NPT, NPT)])


def _agg_body(y_hbm, src_hbm, dst_hbm, out_hbm,
              src_v, dst_v, rows_v, zeros_v, agg_sh, sem):
    c = lax.axis_index("c")
    s = lax.axis_index("s")
    wid = c * NS + s
    for i in range(16):
        for k in range(D // 16):
            zeros_v[i, pl.ds(k * 16, 16)] = jnp.zeros((16,), jnp.float32)

    def zbody(j, carry):
        pltpu.sync_copy(zeros_v, agg_sh.at[pl.ds(s * NPT + j * 16, 16)])
        return carry

    lax.fori_loop(0, NPT // 16, zbody, 0)
    pltpu.sync_copy(src_hbm.at[wid], src_v)
    pltpu.sync_copy(dst_hbm.at[wid], dst_v)
    plsc.subcore_barrier()

    def body(j, carry):
        pltpu.async_copy(y_hbm.at[src_v.at[j]], rows_v, sem).wait()
        pltpu.sync_copy(rows_v, agg_sh.at[dst_v.at[j]], add=True)
        return carry

    lax.fori_loop(0, CPT, body, 0)
    plsc.subcore_barrier()

    def wbody(j, carry):
        pltpu.sync_copy(agg_sh.at[pl.ds(s * NPT + j * 128, 128)],
                        out_hbm.at[c, pl.ds(s * NPT + j * 128, 128)])
        return carry

    lax.fori_loop(0, NPT // 128, wbody, 0)


@functools.lru_cache(maxsize=None)
def _sc_kernels():
    mesh = plsc.VectorSubcoreMesh(
        core_axis_name="c", subcore_axis_name="s",
        num_cores=NC, num_subcores=NS)
    deg_kernel = pl.kernel(
        _deg_body,
        out_type=jax.ShapeDtypeStruct((NC, NP), jnp.float32),
        mesh=mesh,
        scratch_types=[
            pltpu.VMEM((CPT, CH), jnp.int32),
            pltpu.VMEM((CH,), jnp.float32),
            pltpu.VMEM((NPT,), jnp.float32),
            pltpu.VMEM_SHARED((NP,), jnp.float32),
            pltpu.SemaphoreType.DMA,
        ],
    )
    agg_kernel = pl.kernel(
        _agg_body,
        out_type=jax.ShapeDtypeStruct((NC, NP, D), jnp.float32),
        mesh=mesh,
        scratch_types=[
            pltpu.VMEM((CPT, CH), jnp.int32),
            pltpu.VMEM((CPT, CH), jnp.int32),
            pltpu.VMEM((CH, D), jnp.float32),
            pltpu.VMEM((16, D), jnp.float32),
            pltpu.VMEM_SHARED((NP, D), jnp.float32),
            pltpu.SemaphoreType.DMA,
        ],
    )
    return deg_kernel, agg_kernel


def _prep_body(degp_ref, x_ref, dinv_ref, y_ref):
    d = degp_ref[0, :] + degp_ref[1, :] + 1.0
    dinv = lax.rsqrt(d).reshape(-1, 1)
    dinv_ref[...] = dinv
    y_ref[...] = x_ref[...] * dinv


def _prep(degp, xp):
    blk = 128
    grid = NP // blk
    return pl.pallas_call(
        _prep_body,
        grid=(grid,),
        in_specs=[
            pl.BlockSpec((NC, blk), lambda i: (0, i)),
            pl.BlockSpec((blk, D), lambda i: (i, 0)),
        ],
        out_specs=[
            pl.BlockSpec((blk, 1), lambda i: (i, 0)),
            pl.BlockSpec((blk, D), lambda i: (i, 0)),
        ],
        out_shape=[
            jax.ShapeDtypeStruct((NP, 1), jnp.float32),
            jax.ShapeDtypeStruct((NP, D), jnp.float32),
        ],
    )(degp, xp)


def _final_body(p_ref, dinv_ref, y_ref, x0_ref, x_ref, w_ref, o_ref):
    ssum = p_ref[0] + p_ref[1] + y_ref[...]
    agg = ssum * dinv_ref[...]
    h = (1.0 - ALPHA) * agg + ALPHA * x0_ref[...]
    mm = jnp.dot(h, w_ref[...], preferred_element_type=jnp.float32)
    o_ref[...] = x_ref[...] + jnp.maximum(mm, 0.0)


def _final(parts, dinv, y, x0p, xp, W1):
    blk = 512
    grid = NP // blk
    return pl.pallas_call(
        _final_body,
        grid=(grid,),
        in_specs=[
            pl.BlockSpec((NC, blk, D), lambda i: (0, i, 0)),
            pl.BlockSpec((blk, 1), lambda i: (i, 0)),
            pl.BlockSpec((blk, D), lambda i: (i, 0)),
            pl.BlockSpec((blk, D), lambda i: (i, 0)),
            pl.BlockSpec((blk, D), lambda i: (i, 0)),
            pl.BlockSpec((D, D), lambda i: (0, 0)),
        ],
        out_specs=pl.BlockSpec((blk, D), lambda i: (i, 0)),
        out_shape=jax.ShapeDtypeStruct((NP, D), jnp.float32),
    )(parts, dinv, y, x0p, xp, W1)


def kernel(x, x0, edge_index, W1):
    src = edge_index[0]
    dst = edge_index[1]
    pad = jnp.full((EPAD - E,), N, jnp.int32)
    srcp = jnp.concatenate([src, pad]).reshape(NW, CPT, CH)
    dstp = jnp.concatenate([dst, pad]).reshape(NW, CPT, CH)
    xp = jnp.pad(x, ((0, NP - N), (0, 0)))
    x0p = jnp.pad(x0, ((0, NP - N), (0, 0)))

    deg_kernel, agg_kernel = _sc_kernels()
    degp = deg_kernel(dstp)
    dinv, y = _prep(degp, xp)
    parts = agg_kernel(y, srcp, dstp)
    outp = _final(parts, dinv, y, x0p, xp, W1)
    return outp[:N]
